# baseline (speedup 1.0000x reference)
import functools

import jax
import jax.numpy as jnp
from jax import lax
from jax.experimental import pallas as pl
from jax.experimental.pallas import tpu as pltpu
from jax.experimental.pallas import tpu_sc as plsc

NC = 2
NS = 16
NW = NC * NS
L = 16

DMODEL = 512
SEQ = 128
CHUNK = 16
NBUF = 4
LD = 3


def _make_kernel(total, vocab):
    per_w = total // NW
    nchunk = per_w // CHUNK
    pe_chunks = SEQ // CHUNK
    niter = -(-nchunk // NBUF)

    mesh = plsc.VectorSubcoreMesh(core_axis_name="c", subcore_axis_name="s")

    @functools.partial(
        pl.kernel,
        mesh=mesh,
        out_type=jax.ShapeDtypeStruct((total, DMODEL), jnp.float32),
        scratch_types=[
            pltpu.VMEM((nchunk, CHUNK), jnp.int32),
            pltpu.VMEM((SEQ, DMODEL), jnp.float32),
        ]
        + [pltpu.VMEM((CHUNK, DMODEL), jnp.float32) for _ in range(NBUF)]
        + [pltpu.SemaphoreType.DMA for _ in range(2 * NBUF)],
    )
    def emb(idx_hbm, pe_hbm, table_hbm, out_hbm, idx_v, pe_v, *rest):
        bufs = rest[:NBUF]
        gsems = rest[NBUF:2 * NBUF]
        wsems = rest[2 * NBUF:]
        wid = lax.axis_index("s") * NC + lax.axis_index("c")
        pltpu.sync_copy(idx_hbm.at[wid], idx_v)
        pltpu.sync_copy(pe_hbm, pe_v)
        base = wid * per_w

        def gather(c, p):
            return pltpu.make_async_copy(
                table_hbm.at[idx_v.at[c]], bufs[p], gsems[p])

        def write(c, p):
            return pltpu.make_async_copy(
                bufs[p], out_hbm.at[pl.ds(base + c * CHUNK, CHUNK)], wsems[p])

        for c0 in range(LD):
            gather(c0, c0).start()

        def step(i, carry):
            for p in range(NBUF):
                c = i * NBUF + p

                @pl.when(c < nchunk)
                def _():
                    gather(c, p).wait()
                    pos0 = (c % pe_chunks) * CHUNK

                    @plsc.parallel_loop(0, CHUNK)
                    def _(r):
                        pr = pos0 + r
                        for j in range(DMODEL // L):
                            sl = pl.ds(j * L, L)
                            bufs[p][r, sl] = bufs[p][r, sl] + pe_v[pr, sl]

                    write(c, p).start()
                    q = (p + LD) % NBUF

                    @pl.when(c >= NBUF - LD)
                    def _():
                        write(c - (NBUF - LD), q).wait()

                    @pl.when(c + LD < nchunk)
                    def _():
                        gather(c + LD, q).start()

            return carry

        lax.fori_loop(0, niter, step, 0, unroll=False)
        for c in range(nchunk - (NBUF - LD), nchunk):
            write(c, c % NBUF).wait()

    return emb


def kernel(x, table, pe):
    batch, seq = x.shape
    total = batch * seq
    idx = x.reshape(NW, total // NW // CHUNK, CHUNK).astype(jnp.int32)
    pe2d = pe.reshape(pe.shape[1], pe.shape[2])[:seq]
    emb = _make_kernel(total, table.shape[0])
    out = emb(idx, pe2d, table)
    return out.reshape(batch, seq, table.shape[1])


# speedup vs baseline: 1.3405x; 1.3405x over previous
import functools

import jax
import jax.numpy as jnp
from jax import lax
from jax.experimental import pallas as pl
from jax.experimental.pallas import tpu as pltpu
from jax.experimental.pallas import tpu_sc as plsc

NC = 2
NS = 16
NW = NC * NS
L = 16

DMODEL = 512
SEQ = 128
POS_W = SEQ // NW
CHUNK = 32
NBUF = 5
LD = 3


def _make_kernel(total, batch):
    per_w = total // NW
    nchunk = per_w // CHUNK
    cpp = batch // CHUNK
    niter = -(-nchunk // NBUF)

    mesh = plsc.VectorSubcoreMesh(core_axis_name="c", subcore_axis_name="s")

    @functools.partial(
        pl.kernel,
        mesh=mesh,
        out_type=jax.ShapeDtypeStruct((total, DMODEL), jnp.float32),
        scratch_types=[
            pltpu.VMEM((nchunk, CHUNK), jnp.int32),
            pltpu.VMEM((nchunk, CHUNK), jnp.int32),
            pltpu.VMEM((POS_W * DMODEL,), jnp.float32),
        ]
        + [pltpu.VMEM((CHUNK, DMODEL), jnp.float32) for _ in range(NBUF)]
        + [pltpu.SemaphoreType.DMA for _ in range(2 * NBUF)],
    )
    def emb(idx_hbm, oidx_hbm, pe_hbm, table_hbm, out_hbm, idx_v, oidx_v,
            pe_v, *rest):
        bufs = rest[:NBUF]
        gsems = rest[NBUF:2 * NBUF]
        wsems = rest[2 * NBUF:]
        wid = lax.axis_index("s") * NC + lax.axis_index("c")
        pltpu.sync_copy(idx_hbm.at[wid], idx_v)
        pltpu.sync_copy(oidx_hbm.at[wid], oidx_v)
        pltpu.sync_copy(pe_hbm.at[pl.ds(wid * POS_W * DMODEL, POS_W * DMODEL)],
                        pe_v)

        def gather(c, p):
            return pltpu.make_async_copy(
                table_hbm.at[idx_v.at[c]], bufs[p], gsems[p])

        def write(c, p):
            return pltpu.make_async_copy(
                bufs[p], out_hbm.at[oidx_v.at[c]], wsems[p])

        for c0 in range(LD):
            gather(c0, c0).start()

        def step(i, carry):
            for p in range(NBUF):
                c = i * NBUF + p

                @pl.when(c < nchunk)
                def _():
                    gather(c, p).wait()
                    pr = c // cpp

                    pbase = pl.multiple_of(pr * DMODEL, L)

                    for h in range(2):
                        pevs = [
                            pe_v[pl.ds(pbase + (h * 16 + g) * L, L)]
                            for g in range(16)
                        ]

                        @plsc.parallel_loop(0, CHUNK)
                        def _(r):
                            for g in range(16):
                                sl = pl.ds((h * 16 + g) * L, L)
                                bufs[p][r, sl] = bufs[p][r, sl] + pevs[g]

                    write(c, p).start()
                    q = (p + LD) % NBUF

                    @pl.when(c >= NBUF - LD)
                    def _():
                        write(c - (NBUF - LD), q).wait()

                    @pl.when(c + LD < nchunk)
                    def _():
                        gather(c + LD, q).start()

            return carry

        lax.fori_loop(0, niter, step, 0, unroll=False)
        for c in range(nchunk - (NBUF - LD), nchunk):
            write(c, c % NBUF).wait()

    return emb


def kernel(x, table, pe):
    batch, seq = x.shape
    total = batch * seq
    nchunk = total // NW // CHUNK
    idx = x.astype(jnp.int32).T.reshape(NW, nchunk, CHUNK)
    orow = (jnp.arange(batch, dtype=jnp.int32)[None, :] * seq
            + jnp.arange(seq, dtype=jnp.int32)[:, None])
    oidx = orow.reshape(NW, nchunk, CHUNK)
    pe2d = pe.reshape(pe.shape[1] * pe.shape[2])
    emb = _make_kernel(total, batch)
    out = emb(idx, oidx, pe2d, table)
    return out.reshape(batch, seq, table.shape[1])


# speedup vs baseline: 1.3637x; 1.0173x over previous
import functools

import jax
import jax.numpy as jnp
from jax import lax
from jax.experimental import pallas as pl
from jax.experimental.pallas import tpu as pltpu
from jax.experimental.pallas import tpu_sc as plsc

NC = 2
NS = 16
NW = NC * NS
L = 16

DMODEL = 512
SEQ = 128
POS_W = SEQ // NW
CHUNK = 32
NBUF = 6
LD = 4


def _make_kernel(total, batch):
    per_w = total // NW
    nchunk = per_w // CHUNK
    cpp = batch // CHUNK
    niter = -(-nchunk // NBUF)

    mesh = plsc.VectorSubcoreMesh(core_axis_name="c", subcore_axis_name="s")

    @functools.partial(
        pl.kernel,
        mesh=mesh,
        out_type=jax.ShapeDtypeStruct((total, DMODEL), jnp.float32),
        scratch_types=[
            pltpu.VMEM((nchunk, CHUNK), jnp.int32),
            pltpu.VMEM((POS_W * DMODEL,), jnp.float32),
        ]
        + [pltpu.VMEM((CHUNK, DMODEL), jnp.float32) for _ in range(NBUF)]
        + [pltpu.VMEM((CHUNK,), jnp.int32) for _ in range(NBUF)]
        + [pltpu.SemaphoreType.DMA for _ in range(2 * NBUF)],
    )
    def emb(idx_hbm, pe_hbm, table_hbm, out_hbm, idx_v, pe_v, *rest):
        bufs = rest[:NBUF]
        obufs = rest[NBUF:2 * NBUF]
        gsems = rest[2 * NBUF:3 * NBUF]
        wsems = rest[3 * NBUF:]
        wid = lax.axis_index("s") * NC + lax.axis_index("c")
        pltpu.sync_copy(idx_hbm.at[wid], idx_v)
        pltpu.sync_copy(pe_hbm.at[pl.ds(wid * POS_W * DMODEL, POS_W * DMODEL)],
                        pe_v)
        io16 = lax.iota(jnp.int32, L)

        def gather(c, p):
            return pltpu.make_async_copy(
                table_hbm.at[idx_v.at[c]], bufs[p], gsems[p])

        def write(c, p):
            return pltpu.make_async_copy(
                bufs[p], out_hbm.at[obufs[p]], wsems[p])

        for c0 in range(LD):
            gather(c0, c0).start()

        def step(i, carry):
            for p in range(NBUF):
                c = i * NBUF + p

                @pl.when(c < nchunk)
                def _():
                    gather(c, p).wait()
                    pr = c // cpp
                    ostart = ((c % cpp) * CHUNK) * SEQ + wid * POS_W + pr
                    for g in range(CHUNK // L):
                        obufs[p][pl.ds(g * L, L)] = (
                            ostart + (io16 + g * L) * SEQ)

                    pbase = pl.multiple_of(pr * DMODEL, L)

                    for h in range(2):
                        pevs = [
                            pe_v[pl.ds(pbase + (h * 16 + g) * L, L)]
                            for g in range(16)
                        ]

                        @plsc.parallel_loop(0, CHUNK)
                        def _(r):
                            for g in range(16):
                                sl = pl.ds((h * 16 + g) * L, L)
                                bufs[p][r, sl] = bufs[p][r, sl] + pevs[g]

                    write(c, p).start()
                    q = (p + LD) % NBUF

                    @pl.when(c >= NBUF - LD)
                    def _():
                        write(c - (NBUF - LD), q).wait()

                    @pl.when(c + LD < nchunk)
                    def _():
                        gather(c + LD, q).start()

            return carry

        lax.fori_loop(0, niter, step, 0, unroll=False)
        for c in range(nchunk - (NBUF - LD), nchunk):
            write(c, c % NBUF).wait()

    return emb


def kernel(x, table, pe):
    batch, seq = x.shape
    total = batch * seq
    nchunk = total // NW // CHUNK
    idx = x.astype(jnp.int32).T.reshape(NW, nchunk, CHUNK)
    pe2d = pe.reshape(pe.shape[1] * pe.shape[2])
    emb = _make_kernel(total, batch)
    out = emb(idx, pe2d, table)
    return out.reshape(batch, seq, table.shape[1])
